# baseline (speedup 1.0000x reference)
import functools

import jax
import jax.numpy as jnp
import numpy as np
from jax import lax
from jax.experimental import pallas as pl
from jax.experimental.pallas import tpu as pltpu
from jax.experimental.pallas import tpu_sc as plsc

_N = 50000
_H = 16
_K = 15
_D2 = 32
_CIN = 128
_PI = 0.04
_EPS = 1e-5

_BA = 400
_NBA = _N // _BA
_TROWS = (_NBA + 1) * _BA

_BC = 200
_NBC = _N // _BC
_KP = 16

_BD = 1000
_NBD = _N // _BD

_ROWS = _N * _H
_NW = 32
_PERW = _ROWS // _NW
_CHUNK = 1000
_NIT = _PERW // _CHUNK


def _leaky(v):
    return jnp.where(v >= 0, v, 0.2 * v)


def _a_body(x_ref, pos_ref, w1_ref, tab_ref, st_ref):
    i = pl.program_id(0)
    y1 = jnp.dot(x_ref[...], w1_ref[...], preferred_element_type=jnp.float32)
    real = i < _NBA
    row = jnp.concatenate(
        [pos_ref[...], jnp.zeros((_BA, 1), jnp.float32), y1,
         jnp.zeros((_BA, 12), jnp.float32)], axis=1)
    shadow = jnp.concatenate(
        [jnp.full((_BA, 3), 1e6, jnp.float32),
         jnp.zeros((_BA, 45), jnp.float32)], axis=1)
    tab_ref[...] = jnp.where(real, row, shadow)

    @pl.when(i == 0)
    def _():
        st_ref[...] = jnp.zeros_like(st_ref)

    @pl.when(real)
    def _():
        st_ref[0:1, 0:_D2] += jnp.sum(y1, axis=0, keepdims=True)
        st_ref[1:2, 0:_D2] += jnp.sum(y1 * y1, axis=0, keepdims=True)


def _run_a(x, pos, w1):
    return pl.pallas_call(
        _a_body,
        grid=(_NBA + 1,),
        in_specs=[
            pl.BlockSpec((_BA, _CIN), lambda i: (jnp.minimum(i, _NBA - 1), 0)),
            pl.BlockSpec((_BA, 3), lambda i: (jnp.minimum(i, _NBA - 1), 0)),
            pl.BlockSpec((_CIN, _D2), lambda i: (0, 0)),
        ],
        out_specs=[
            pl.BlockSpec((_BA, 48), lambda i: (i, 0)),
            pl.BlockSpec((8, 128), lambda i: (0, 0)),
        ],
        out_shape=[
            jax.ShapeDtypeStruct((_TROWS, 48), jnp.float32),
            jax.ShapeDtypeStruct((8, 128), jnp.float32),
        ],
    )(x, pos, w1)


def _gather_body(tab_hbm, idx_hbm, out_hbm, idx_v, rows_v, sem):
    wid = lax.axis_index("s") * 2 + lax.axis_index("c")
    base = wid * _PERW

    def body(it, carry):
        off = base + it * _CHUNK
        pltpu.sync_copy(idx_hbm.at[pl.ds(off, _CHUNK)], idx_v)
        pltpu.async_copy(tab_hbm.at[idx_v], rows_v, sem).wait()
        pltpu.sync_copy(rows_v, out_hbm.at[pl.ds(off, _CHUNK)])
        return carry

    lax.fori_loop(0, _NIT, body, 0)


def _run_gather(tab, idx_flat):
    mesh = plsc.VectorSubcoreMesh(core_axis_name="c", subcore_axis_name="s")
    f = functools.partial(
        pl.kernel,
        mesh=mesh,
        compiler_params=pltpu.CompilerParams(use_tc_tiling_on_sc=False),
        out_type=jax.ShapeDtypeStruct((_ROWS, 48), jnp.float32),
        scratch_types=[
            pltpu.VMEM((_CHUNK,), jnp.int32),
            pltpu.VMEM((_CHUNK, 48), jnp.float32),
            pltpu.SemaphoreType.DMA,
        ],
    )(_gather_body)
    return f(tab, idx_flat)


def _c_body(pos_ref, g_ref, st1_ref, g1_ref, b1_ref, sel_ref, t64_ref,
            tf_ref, geo_ref, kn_ref, kpf_ref, y2_ref, st2_ref, wf_ref,
            wf2_ref):
    i = pl.program_id(0)
    mean = st1_ref[0:1, 0:_D2] * (1.0 / _N)
    ex2 = st1_ref[1:2, 0:_D2] * (1.0 / _N)
    var = ex2 - mean * mean
    a = lax.rsqrt(var + _EPS) * g1_ref[...]
    c = b1_ref[...] - mean * a
    qpos = pos_ref[...]
    g2v = g_ref[...]
    ktT = geo_ref[0:3, 0:_KP] * (-0.5)
    kn16 = kn_ref[0:1, 0:_KP]
    ws = []
    for h in range(_H):
        d = g_ref[:, 48 * h:48 * h + 3] - qpos
        sqn = jnp.sum(d * d, axis=1, keepdims=True)
        dt = (d[:, 0:1] * ktT[0:1] + d[:, 1:2] * ktT[1:2]
              + d[:, 2:3] * ktT[2:3])
        sqd_h = jnp.maximum(sqn - 2.0 * dt + kn16, 0.0)
        ws.append(jnp.maximum(1.0 - jnp.sqrt(sqd_h) * (1.0 / _PI), 0.0))
    w = jnp.concatenate(ws, axis=1)
    a768 = jnp.dot(a, tf_ref[...], preferred_element_type=jnp.float32)
    c768 = jnp.dot(c, tf_ref[...], preferred_element_type=jnp.float32)
    f_all = _leaky(g2v * a768 + c768)

    def _prod(h):
        f_h = f_all[:, 48 * h + 4:48 * h + 36]
        return jnp.concatenate(
            [w[:, _KP * h + k:_KP * h + k + 1] * f_h for k in range(_KP)],
            axis=1)

    wf_ref[...] = _prod(0) + _prod(1)
    wf2_ref[...] = _prod(2) + _prod(3)
    for j in range(1, 4):
        wf_ref[...] += _prod(4 * j) + _prod(4 * j + 1)
        wf2_ref[...] += _prod(4 * j + 2) + _prod(4 * j + 3)
    y2 = jnp.dot(wf_ref[...] + wf2_ref[...], kpf_ref[...],
                 preferred_element_type=jnp.float32)
    y2_ref[...] = y2

    @pl.when(i == 0)
    def _():
        st2_ref[...] = jnp.zeros_like(st2_ref)

    st2_ref[0:1, 0:_D2] += jnp.sum(y2, axis=0, keepdims=True)
    st2_ref[1:2, 0:_D2] += jnp.sum(y2 * y2, axis=0, keepdims=True)


def _run_c(pos, g2, st1, g1, b1, sel, t64, tf, geo, kn, kpf):
    return pl.pallas_call(
        _c_body,
        grid=(_NBC,),
        in_specs=[
            pl.BlockSpec((_BC, 3), lambda i: (i, 0)),
            pl.BlockSpec((_BC, _H * 48), lambda i: (i, 0)),
            pl.BlockSpec((8, 128), lambda i: (0, 0)),
            pl.BlockSpec((1, _D2), lambda i: (0, 0)),
            pl.BlockSpec((1, _D2), lambda i: (0, 0)),
            pl.BlockSpec((768, 64), lambda i: (0, 0)),
            pl.BlockSpec((3, 64), lambda i: (0, 0)),
            pl.BlockSpec((_D2, 768), lambda i: (0, 0)),
            pl.BlockSpec((198, _KP * _H), lambda i: (0, 0)),
            pl.BlockSpec((1, _KP * _H), lambda i: (0, 0)),
            pl.BlockSpec((_KP * _D2, _D2), lambda i: (0, 0)),
        ],
        out_specs=[
            pl.BlockSpec((_BC, _D2), lambda i: (i, 0)),
            pl.BlockSpec((8, 128), lambda i: (0, 0)),
        ],
        out_shape=[
            jax.ShapeDtypeStruct((_N, _D2), jnp.float32),
            jax.ShapeDtypeStruct((8, 128), jnp.float32),
        ],
        scratch_shapes=[pltpu.VMEM((_BC, _KP * _D2), jnp.float32),
                        pltpu.VMEM((_BC, _KP * _D2), jnp.float32)],
    )(pos, g2, st1, g1, b1, sel, t64, tf, geo, kn, kpf)


def _d_body(y2_ref, st2_ref, gkp_ref, bkp_ref, w2_ref, y3_ref, st3_ref):
    i = pl.program_id(0)
    mean = st2_ref[0:1, 0:_D2] * (1.0 / _N)
    ex2 = st2_ref[1:2, 0:_D2] * (1.0 / _N)
    var = ex2 - mean * mean
    a = lax.rsqrt(var + _EPS) * gkp_ref[...]
    c = bkp_ref[...] - mean * a
    h2 = _leaky(y2_ref[...] * a + c)
    y3 = jnp.dot(h2, w2_ref[...], preferred_element_type=jnp.float32)
    y3_ref[...] = y3

    @pl.when(i == 0)
    def _():
        st3_ref[...] = jnp.zeros_like(st3_ref)

    st3_ref[0:1, :] += jnp.sum(y3, axis=0, keepdims=True)
    st3_ref[1:2, :] += jnp.sum(y3 * y3, axis=0, keepdims=True)


def _run_d(y2, st2, gkp, bkp, w2):
    return pl.pallas_call(
        _d_body,
        grid=(_NBD,),
        in_specs=[
            pl.BlockSpec((_BD, _D2), lambda i: (i, 0)),
            pl.BlockSpec((8, 128), lambda i: (0, 0)),
            pl.BlockSpec((1, _D2), lambda i: (0, 0)),
            pl.BlockSpec((1, _D2), lambda i: (0, 0)),
            pl.BlockSpec((_D2, _CIN), lambda i: (0, 0)),
        ],
        out_specs=[
            pl.BlockSpec((_BD, _CIN), lambda i: (i, 0)),
            pl.BlockSpec((8, 128), lambda i: (0, 0)),
        ],
        out_shape=[
            jax.ShapeDtypeStruct((_N, _CIN), jnp.float32),
            jax.ShapeDtypeStruct((8, 128), jnp.float32),
        ],
    )(y2, st2, gkp, bkp, w2)


def _e_body(y3_ref, st3_ref, g2_ref, b2_ref, x_ref, out_ref):
    mean = st3_ref[0:1, :] * (1.0 / _N)
    ex2 = st3_ref[1:2, :] * (1.0 / _N)
    var = ex2 - mean * mean
    a = lax.rsqrt(var + _EPS) * g2_ref[...]
    c = b2_ref[...] - mean * a
    out_ref[...] = _leaky(y3_ref[...] * a + c) + x_ref[...]


def _run_e(y3, st3, g2v, b2v, x):
    return pl.pallas_call(
        _e_body,
        grid=(_NBD,),
        in_specs=[
            pl.BlockSpec((_BD, _CIN), lambda i: (i, 0)),
            pl.BlockSpec((8, 128), lambda i: (0, 0)),
            pl.BlockSpec((1, _CIN), lambda i: (0, 0)),
            pl.BlockSpec((1, _CIN), lambda i: (0, 0)),
            pl.BlockSpec((_BD, _CIN), lambda i: (i, 0)),
        ],
        out_specs=pl.BlockSpec((_BD, _CIN), lambda i: (i, 0)),
        out_shape=jax.ShapeDtypeStruct((_N, _CIN), jnp.float32),
    )(y3, st3, g2v, b2v, x)


_SEL_NP = np.zeros((768, 64), np.float32)
_T64_NP = np.zeros((3, 64), np.float32)
_TF_NP = np.zeros((_D2, 768), np.float32)
for _h in range(_H):
    for _c in range(4):
        _SEL_NP[48 * _h + _c, 4 * _h + _c] = 1.0
    for _c in range(3):
        _T64_NP[_c, 4 * _h + _c] = 1.0
    for _c in range(_D2):
        _TF_NP[_c, 48 * _h + 4 + _c] = 1.0


def kernel(x, pos, idx_neighboors, w_unary1, gamma1, beta1, kp_weight,
           K_points, gamma_kp, beta_kp, w_unary2, gamma2, beta2):
    idx_flat = idx_neighboors.reshape(-1).astype(jnp.int32)
    tab, st1 = _run_a(x, pos, w_unary1)
    g = _run_gather(tab, idx_flat)
    g2 = g.reshape(_N, _H * 48)
    eye16 = jnp.eye(_H, dtype=jnp.float32)
    blkK = jnp.zeros((4, _KP), jnp.float32).at[0:3, 0:_K].set(K_points.T)
    blk1 = jnp.zeros((4, _KP), jnp.float32).at[0:3, 0:_K].set(1.0)
    geo = jnp.concatenate([
        jnp.kron(eye16, -2.0 * blkK),
        jnp.kron(eye16, blk1),
        jnp.kron(eye16, -2.0 * blk1),
        jnp.tile(jnp.zeros((3, _KP), jnp.float32).at[:, 0:_K]
                 .set(2.0 * K_points.T), (1, _H)),
        jnp.tile(jnp.zeros((3, _KP), jnp.float32).at[:, 0:_K].set(1.0),
                 (1, _H)),
    ], axis=0)
    kn16 = (jnp.zeros((_KP,), jnp.float32)
            .at[0:_K].set(jnp.sum(K_points * K_points, axis=1))
            .at[_K].set(1e12))
    kn = jnp.tile(kn16, _H)[None]
    kpf = jnp.concatenate(
        [kp_weight.reshape(_K * _D2, _D2),
         jnp.zeros((_D2, _D2), jnp.float32)], axis=0)
    y2, st2 = _run_c(pos, g2, st1, gamma1[None], beta1[None],
                     jnp.asarray(_SEL_NP), jnp.asarray(_T64_NP),
                     jnp.asarray(_TF_NP), geo, kn, kpf)
    y3, st3 = _run_d(y2, st2, gamma_kp[None], beta_kp[None], w_unary2)
    return _run_e(y3, st3, gamma2[None], beta2[None], x)


# speedup vs baseline: 1.5122x; 1.5122x over previous
import functools

import jax
import jax.numpy as jnp
import numpy as np
from jax import lax
from jax.experimental import pallas as pl
from jax.experimental.pallas import tpu as pltpu
from jax.experimental.pallas import tpu_sc as plsc

_N = 50000
_H = 16
_K = 15
_D2 = 32
_CIN = 128
_PI = 0.04
_EPS = 1e-5

_BA = 400
_NBA = _N // _BA
_TROWS = (_NBA + 1) * _BA

_BC = 400
_NBC = _N // _BC
_KP = 16

_BD = 1000
_NBD = _N // _BD

_ROWS = _N * _H
_NW = 32
_PERW = _ROWS // _NW
_CHUNK = 1000
_NIT = _PERW // _CHUNK


def _leaky(v):
    return jnp.where(v >= 0, v, 0.2 * v)


def _a_body(x_ref, pos_ref, w1_ref, tab_ref, st_ref):
    i = pl.program_id(0)
    y1 = jnp.dot(x_ref[...], w1_ref[...], preferred_element_type=jnp.float32)
    real = i < _NBA
    row = jnp.concatenate(
        [pos_ref[...], jnp.zeros((_BA, 1), jnp.float32), y1,
         jnp.zeros((_BA, 12), jnp.float32)], axis=1)
    shadow = jnp.concatenate(
        [jnp.full((_BA, 3), 1e6, jnp.float32),
         jnp.zeros((_BA, 45), jnp.float32)], axis=1)
    tab_ref[...] = jnp.where(real, row, shadow)

    @pl.when(i == 0)
    def _():
        st_ref[...] = jnp.zeros_like(st_ref)

    @pl.when(real)
    def _():
        st_ref[0:1, 0:_D2] += jnp.sum(y1, axis=0, keepdims=True)
        st_ref[1:2, 0:_D2] += jnp.sum(y1 * y1, axis=0, keepdims=True)


def _run_a(x, pos, w1):
    return pl.pallas_call(
        _a_body,
        grid=(_NBA + 1,),
        in_specs=[
            pl.BlockSpec((_BA, _CIN), lambda i: (jnp.minimum(i, _NBA - 1), 0)),
            pl.BlockSpec((_BA, 3), lambda i: (jnp.minimum(i, _NBA - 1), 0)),
            pl.BlockSpec((_CIN, _D2), lambda i: (0, 0)),
        ],
        out_specs=[
            pl.BlockSpec((_BA, 48), lambda i: (i, 0)),
            pl.BlockSpec((8, 128), lambda i: (0, 0)),
        ],
        out_shape=[
            jax.ShapeDtypeStruct((_TROWS, 48), jnp.float32),
            jax.ShapeDtypeStruct((8, 128), jnp.float32),
        ],
    )(x, pos, w1)


def _gather_body(tab_hbm, idx_hbm, out_hbm, idx_v, rows_v, sem):
    wid = lax.axis_index("s") * 2 + lax.axis_index("c")
    base = wid * _PERW

    def body(it, carry):
        off = base + it * _CHUNK
        pltpu.sync_copy(idx_hbm.at[pl.ds(off, _CHUNK)], idx_v)
        pltpu.async_copy(tab_hbm.at[idx_v], rows_v, sem).wait()
        pltpu.sync_copy(rows_v, out_hbm.at[pl.ds(off, _CHUNK)])
        return carry

    lax.fori_loop(0, _NIT, body, 0)


def _run_gather(tab, idx_flat):
    mesh = plsc.VectorSubcoreMesh(core_axis_name="c", subcore_axis_name="s")
    f = functools.partial(
        pl.kernel,
        mesh=mesh,
        compiler_params=pltpu.CompilerParams(use_tc_tiling_on_sc=False),
        out_type=jax.ShapeDtypeStruct((_ROWS, 48), jnp.float32),
        scratch_types=[
            pltpu.VMEM((_CHUNK,), jnp.int32),
            pltpu.VMEM((_CHUNK, 48), jnp.float32),
            pltpu.SemaphoreType.DMA,
        ],
    )(_gather_body)
    return f(tab, idx_flat)


def _c_body(pos_ref, g_ref, st1_ref, g1_ref, b1_ref, sel_ref, t64_ref,
            tf_ref, geo_ref, kn_ref, kpf_ref, y2_ref, st2_ref, wf_ref,
            wf2_ref):
    i = pl.program_id(0)
    mean = st1_ref[0:1, 0:_D2] * (1.0 / _N)
    ex2 = st1_ref[1:2, 0:_D2] * (1.0 / _N)
    var = ex2 - mean * mean
    a = lax.rsqrt(var + _EPS) * g1_ref[...]
    c = b1_ref[...] - mean * a
    qpos = pos_ref[...]
    g2v = g_ref[...]
    ktT = geo_ref[0:3, 0:_KP] * (-0.5)
    kn16 = kn_ref[0:1, 0:_KP]
    ws = []
    for h in range(_H):
        d = g_ref[:, 48 * h:48 * h + 3] - qpos
        sqn = jnp.sum(d * d, axis=1, keepdims=True)
        dt = (d[:, 0:1] * ktT[0:1] + d[:, 1:2] * ktT[1:2]
              + d[:, 2:3] * ktT[2:3])
        sqd_h = jnp.maximum(sqn - 2.0 * dt + kn16, 0.0)
        ws.append(jnp.maximum(1.0 - jnp.sqrt(sqd_h) * (1.0 / _PI), 0.0))
    w = jnp.concatenate(ws, axis=1)
    a768 = jnp.dot(a, tf_ref[...], preferred_element_type=jnp.float32)
    c768 = jnp.dot(c, tf_ref[...], preferred_element_type=jnp.float32)
    f_all = _leaky(g2v * a768 + c768)

    def _prod(h):
        f_h = f_all[:, 48 * h + 4:48 * h + 36]
        wexp = jnp.concatenate(
            [jnp.broadcast_to(w[:, _KP * h + k:_KP * h + k + 1], (_BC, _D2))
             for k in range(_KP)], axis=1)
        tile = jnp.concatenate([f_h] * _KP, axis=1)
        return wexp * tile

    wf_ref[...] = _prod(0) + _prod(1)
    wf2_ref[...] = _prod(2) + _prod(3)
    for j in range(1, 4):
        wf_ref[...] += _prod(4 * j) + _prod(4 * j + 1)
        wf2_ref[...] += _prod(4 * j + 2) + _prod(4 * j + 3)
    y2 = jnp.dot(wf_ref[...] + wf2_ref[...], kpf_ref[...],
                 preferred_element_type=jnp.float32)
    y2_ref[...] = y2

    @pl.when(i == 0)
    def _():
        st2_ref[...] = jnp.zeros_like(st2_ref)

    st2_ref[0:1, 0:_D2] += jnp.sum(y2, axis=0, keepdims=True)
    st2_ref[1:2, 0:_D2] += jnp.sum(y2 * y2, axis=0, keepdims=True)


def _run_c(pos, g2, st1, g1, b1, sel, t64, tf, geo, kn, kpf):
    return pl.pallas_call(
        _c_body,
        grid=(_NBC,),
        in_specs=[
            pl.BlockSpec((_BC, 3), lambda i: (i, 0)),
            pl.BlockSpec((_BC, _H * 48), lambda i: (i, 0)),
            pl.BlockSpec((8, 128), lambda i: (0, 0)),
            pl.BlockSpec((1, _D2), lambda i: (0, 0)),
            pl.BlockSpec((1, _D2), lambda i: (0, 0)),
            pl.BlockSpec((768, 64), lambda i: (0, 0)),
            pl.BlockSpec((3, 64), lambda i: (0, 0)),
            pl.BlockSpec((_D2, 768), lambda i: (0, 0)),
            pl.BlockSpec((198, _KP * _H), lambda i: (0, 0)),
            pl.BlockSpec((1, _KP * _H), lambda i: (0, 0)),
            pl.BlockSpec((_KP * _D2, _D2), lambda i: (0, 0)),
        ],
        out_specs=[
            pl.BlockSpec((_BC, _D2), lambda i: (i, 0)),
            pl.BlockSpec((8, 128), lambda i: (0, 0)),
        ],
        out_shape=[
            jax.ShapeDtypeStruct((_N, _D2), jnp.float32),
            jax.ShapeDtypeStruct((8, 128), jnp.float32),
        ],
        scratch_shapes=[pltpu.VMEM((_BC, _KP * _D2), jnp.float32),
                        pltpu.VMEM((_BC, _KP * _D2), jnp.float32)],
    )(pos, g2, st1, g1, b1, sel, t64, tf, geo, kn, kpf)


def _d_body(y2_ref, st2_ref, gkp_ref, bkp_ref, w2_ref, y3_ref, st3_ref):
    i = pl.program_id(0)
    mean = st2_ref[0:1, 0:_D2] * (1.0 / _N)
    ex2 = st2_ref[1:2, 0:_D2] * (1.0 / _N)
    var = ex2 - mean * mean
    a = lax.rsqrt(var + _EPS) * gkp_ref[...]
    c = bkp_ref[...] - mean * a
    h2 = _leaky(y2_ref[...] * a + c)
    y3 = jnp.dot(h2, w2_ref[...], preferred_element_type=jnp.float32)
    y3_ref[...] = y3

    @pl.when(i == 0)
    def _():
        st3_ref[...] = jnp.zeros_like(st3_ref)

    st3_ref[0:1, :] += jnp.sum(y3, axis=0, keepdims=True)
    st3_ref[1:2, :] += jnp.sum(y3 * y3, axis=0, keepdims=True)


def _run_d(y2, st2, gkp, bkp, w2):
    return pl.pallas_call(
        _d_body,
        grid=(_NBD,),
        in_specs=[
            pl.BlockSpec((_BD, _D2), lambda i: (i, 0)),
            pl.BlockSpec((8, 128), lambda i: (0, 0)),
            pl.BlockSpec((1, _D2), lambda i: (0, 0)),
            pl.BlockSpec((1, _D2), lambda i: (0, 0)),
            pl.BlockSpec((_D2, _CIN), lambda i: (0, 0)),
        ],
        out_specs=[
            pl.BlockSpec((_BD, _CIN), lambda i: (i, 0)),
            pl.BlockSpec((8, 128), lambda i: (0, 0)),
        ],
        out_shape=[
            jax.ShapeDtypeStruct((_N, _CIN), jnp.float32),
            jax.ShapeDtypeStruct((8, 128), jnp.float32),
        ],
    )(y2, st2, gkp, bkp, w2)


def _e_body(y3_ref, st3_ref, g2_ref, b2_ref, x_ref, out_ref):
    mean = st3_ref[0:1, :] * (1.0 / _N)
    ex2 = st3_ref[1:2, :] * (1.0 / _N)
    var = ex2 - mean * mean
    a = lax.rsqrt(var + _EPS) * g2_ref[...]
    c = b2_ref[...] - mean * a
    out_ref[...] = _leaky(y3_ref[...] * a + c) + x_ref[...]


def _run_e(y3, st3, g2v, b2v, x):
    return pl.pallas_call(
        _e_body,
        grid=(_NBD,),
        in_specs=[
            pl.BlockSpec((_BD, _CIN), lambda i: (i, 0)),
            pl.BlockSpec((8, 128), lambda i: (0, 0)),
            pl.BlockSpec((1, _CIN), lambda i: (0, 0)),
            pl.BlockSpec((1, _CIN), lambda i: (0, 0)),
            pl.BlockSpec((_BD, _CIN), lambda i: (i, 0)),
        ],
        out_specs=pl.BlockSpec((_BD, _CIN), lambda i: (i, 0)),
        out_shape=jax.ShapeDtypeStruct((_N, _CIN), jnp.float32),
    )(y3, st3, g2v, b2v, x)


_SEL_NP = np.zeros((768, 64), np.float32)
_T64_NP = np.zeros((3, 64), np.float32)
_TF_NP = np.zeros((_D2, 768), np.float32)
for _h in range(_H):
    for _c in range(4):
        _SEL_NP[48 * _h + _c, 4 * _h + _c] = 1.0
    for _c in range(3):
        _T64_NP[_c, 4 * _h + _c] = 1.0
    for _c in range(_D2):
        _TF_NP[_c, 48 * _h + 4 + _c] = 1.0


def kernel(x, pos, idx_neighboors, w_unary1, gamma1, beta1, kp_weight,
           K_points, gamma_kp, beta_kp, w_unary2, gamma2, beta2):
    idx_flat = idx_neighboors.reshape(-1).astype(jnp.int32)
    tab, st1 = _run_a(x, pos, w_unary1)
    g = _run_gather(tab, idx_flat)
    g2 = g.reshape(_N, _H * 48)
    eye16 = jnp.eye(_H, dtype=jnp.float32)
    blkK = jnp.zeros((4, _KP), jnp.float32).at[0:3, 0:_K].set(K_points.T)
    blk1 = jnp.zeros((4, _KP), jnp.float32).at[0:3, 0:_K].set(1.0)
    geo = jnp.concatenate([
        jnp.kron(eye16, -2.0 * blkK),
        jnp.kron(eye16, blk1),
        jnp.kron(eye16, -2.0 * blk1),
        jnp.tile(jnp.zeros((3, _KP), jnp.float32).at[:, 0:_K]
                 .set(2.0 * K_points.T), (1, _H)),
        jnp.tile(jnp.zeros((3, _KP), jnp.float32).at[:, 0:_K].set(1.0),
                 (1, _H)),
    ], axis=0)
    kn16 = (jnp.zeros((_KP,), jnp.float32)
            .at[0:_K].set(jnp.sum(K_points * K_points, axis=1))
            .at[_K].set(1e12))
    kn = jnp.tile(kn16, _H)[None]
    kpf = jnp.concatenate(
        [kp_weight.reshape(_K * _D2, _D2),
         jnp.zeros((_D2, _D2), jnp.float32)], axis=0)
    y2, st2 = _run_c(pos, g2, st1, gamma1[None], beta1[None],
                     jnp.asarray(_SEL_NP), jnp.asarray(_T64_NP),
                     jnp.asarray(_TF_NP), geo, kn, kpf)
    y3, st3 = _run_d(y2, st2, gamma_kp[None], beta_kp[None], w_unary2)
    return _run_e(y3, st3, gamma2[None], beta2[None], x)


# speedup vs baseline: 1.5158x; 1.0024x over previous
import functools

import jax
import jax.numpy as jnp
import numpy as np
from jax import lax
from jax.experimental import pallas as pl
from jax.experimental.pallas import tpu as pltpu
from jax.experimental.pallas import tpu_sc as plsc

_N = 50000
_H = 16
_K = 15
_D2 = 32
_CIN = 128
_PI = 0.04
_EPS = 1e-5

_BA = 400
_NBA = _N // _BA
_TROWS = (_NBA + 1) * _BA

_BC = 400
_NBC = _N // _BC
_KP = 16

_BD = 1000
_NBD = _N // _BD

_ROWS = _N * _H
_NW = 32
_PERW = _ROWS // _NW
_CHUNK = 1000
_NIT = _PERW // _CHUNK


def _leaky(v):
    return jnp.where(v >= 0, v, 0.2 * v)


def _a_body(x_ref, pos_ref, w1_ref, tab_ref, st_ref):
    i = pl.program_id(0)
    y1 = jnp.dot(x_ref[...], w1_ref[...], preferred_element_type=jnp.float32)
    real = i < _NBA
    row = jnp.concatenate(
        [pos_ref[...], jnp.zeros((_BA, 1), jnp.float32), y1,
         jnp.zeros((_BA, 12), jnp.float32)], axis=1)
    shadow = jnp.concatenate(
        [jnp.full((_BA, 3), 1e6, jnp.float32),
         jnp.zeros((_BA, 45), jnp.float32)], axis=1)
    tab_ref[...] = jnp.where(real, row, shadow)

    @pl.when(i == 0)
    def _():
        st_ref[...] = jnp.zeros_like(st_ref)

    @pl.when(real)
    def _():
        st_ref[0:1, 0:_D2] += jnp.sum(y1, axis=0, keepdims=True)
        st_ref[1:2, 0:_D2] += jnp.sum(y1 * y1, axis=0, keepdims=True)


def _run_a(x, pos, w1):
    return pl.pallas_call(
        _a_body,
        grid=(_NBA + 1,),
        in_specs=[
            pl.BlockSpec((_BA, _CIN), lambda i: (jnp.minimum(i, _NBA - 1), 0)),
            pl.BlockSpec((_BA, 3), lambda i: (jnp.minimum(i, _NBA - 1), 0)),
            pl.BlockSpec((_CIN, _D2), lambda i: (0, 0)),
        ],
        out_specs=[
            pl.BlockSpec((_BA, 48), lambda i: (i, 0)),
            pl.BlockSpec((8, 128), lambda i: (0, 0)),
        ],
        out_shape=[
            jax.ShapeDtypeStruct((_TROWS, 48), jnp.float32),
            jax.ShapeDtypeStruct((8, 128), jnp.float32),
        ],
    )(x, pos, w1)


def _gather_body(tab_hbm, idx_hbm, out_hbm, idx_v, rows_v, sem):
    wid = lax.axis_index("s") * 2 + lax.axis_index("c")
    base = wid * _PERW

    def body(it, carry):
        off = base + it * _CHUNK
        pltpu.sync_copy(idx_hbm.at[pl.ds(off, _CHUNK)], idx_v)
        pltpu.async_copy(tab_hbm.at[idx_v], rows_v, sem).wait()
        pltpu.sync_copy(rows_v, out_hbm.at[pl.ds(off, _CHUNK)])
        return carry

    lax.fori_loop(0, _NIT, body, 0)


def _run_gather(tab, idx_flat):
    mesh = plsc.VectorSubcoreMesh(core_axis_name="c", subcore_axis_name="s")
    f = functools.partial(
        pl.kernel,
        mesh=mesh,
        compiler_params=pltpu.CompilerParams(use_tc_tiling_on_sc=False),
        out_type=jax.ShapeDtypeStruct((_ROWS, 48), jnp.float32),
        scratch_types=[
            pltpu.VMEM((_CHUNK,), jnp.int32),
            pltpu.VMEM((_CHUNK, 48), jnp.float32),
            pltpu.SemaphoreType.DMA,
        ],
    )(_gather_body)
    return f(tab, idx_flat)


def _c_body(pos_ref, g_ref, st1_ref, g1_ref, b1_ref, sel_ref, t64_ref,
            tf_ref, geo_ref, kn_ref, kpf_ref, y2_ref, st2_ref, wf_ref,
            wf2_ref):
    i = pl.program_id(0)
    mean = st1_ref[0:1, 0:_D2] * (1.0 / _N)
    ex2 = st1_ref[1:2, 0:_D2] * (1.0 / _N)
    var = ex2 - mean * mean
    a = lax.rsqrt(var + _EPS) * g1_ref[...]
    c = b1_ref[...] - mean * a
    qpos = pos_ref[...]
    g2v = g_ref[...]
    ktT = geo_ref[0:3, 0:_KP] * (-0.5)
    kn16 = kn_ref[0:1, 0:_KP]
    ws = []
    for h in range(_H):
        d = g_ref[:, 48 * h:48 * h + 3] - qpos
        sqn = jnp.sum(d * d, axis=1, keepdims=True)
        dt = (d[:, 0:1] * ktT[0:1] + d[:, 1:2] * ktT[1:2]
              + d[:, 2:3] * ktT[2:3])
        sqd_h = jnp.maximum(sqn - 2.0 * dt + kn16, 0.0)
        ws.append(jnp.maximum(1.0 - jnp.sqrt(sqd_h) * (1.0 / _PI), 0.0))
    w = jnp.concatenate(ws, axis=1)
    a768 = jnp.dot(a, tf_ref[...], preferred_element_type=jnp.float32)
    c768 = jnp.dot(c, tf_ref[...], preferred_element_type=jnp.float32)
    f_all = _leaky(g2v * a768 + c768)

    def _prod(h):
        f_h = f_all[:, 48 * h + 4:48 * h + 36]
        wexp = jnp.concatenate(
            [jnp.broadcast_to(w[:, _KP * h + k:_KP * h + k + 1], (_BC, _D2))
             for k in range(_KP)], axis=1)
        tile = jnp.concatenate([f_h] * _KP, axis=1)
        return wexp * tile

    wf_ref[...] = (_prod(0) + _prod(1)) + (_prod(2) + _prod(3))
    wf2_ref[...] = (_prod(4) + _prod(5)) + (_prod(6) + _prod(7))
    wf_ref[...] += (_prod(8) + _prod(9)) + (_prod(10) + _prod(11))
    wf2_ref[...] += (_prod(12) + _prod(13)) + (_prod(14) + _prod(15))
    y2 = jnp.dot(wf_ref[...] + wf2_ref[...], kpf_ref[...],
                 preferred_element_type=jnp.float32)
    y2_ref[...] = y2

    @pl.when(i == 0)
    def _():
        st2_ref[...] = jnp.zeros_like(st2_ref)

    st2_ref[0:1, 0:_D2] += jnp.sum(y2, axis=0, keepdims=True)
    st2_ref[1:2, 0:_D2] += jnp.sum(y2 * y2, axis=0, keepdims=True)


def _run_c(pos, g2, st1, g1, b1, sel, t64, tf, geo, kn, kpf):
    return pl.pallas_call(
        _c_body,
        grid=(_NBC,),
        in_specs=[
            pl.BlockSpec((_BC, 3), lambda i: (i, 0)),
            pl.BlockSpec((_BC, _H * 48), lambda i: (i, 0)),
            pl.BlockSpec((8, 128), lambda i: (0, 0)),
            pl.BlockSpec((1, _D2), lambda i: (0, 0)),
            pl.BlockSpec((1, _D2), lambda i: (0, 0)),
            pl.BlockSpec((768, 64), lambda i: (0, 0)),
            pl.BlockSpec((3, 64), lambda i: (0, 0)),
            pl.BlockSpec((_D2, 768), lambda i: (0, 0)),
            pl.BlockSpec((198, _KP * _H), lambda i: (0, 0)),
            pl.BlockSpec((1, _KP * _H), lambda i: (0, 0)),
            pl.BlockSpec((_KP * _D2, _D2), lambda i: (0, 0)),
        ],
        out_specs=[
            pl.BlockSpec((_BC, _D2), lambda i: (i, 0)),
            pl.BlockSpec((8, 128), lambda i: (0, 0)),
        ],
        out_shape=[
            jax.ShapeDtypeStruct((_N, _D2), jnp.float32),
            jax.ShapeDtypeStruct((8, 128), jnp.float32),
        ],
        scratch_shapes=[pltpu.VMEM((_BC, _KP * _D2), jnp.float32),
                        pltpu.VMEM((_BC, _KP * _D2), jnp.float32)],
    )(pos, g2, st1, g1, b1, sel, t64, tf, geo, kn, kpf)


def _d_body(y2_ref, st2_ref, gkp_ref, bkp_ref, w2_ref, y3_ref, st3_ref):
    i = pl.program_id(0)
    mean = st2_ref[0:1, 0:_D2] * (1.0 / _N)
    ex2 = st2_ref[1:2, 0:_D2] * (1.0 / _N)
    var = ex2 - mean * mean
    a = lax.rsqrt(var + _EPS) * gkp_ref[...]
    c = bkp_ref[...] - mean * a
    h2 = _leaky(y2_ref[...] * a + c)
    y3 = jnp.dot(h2, w2_ref[...], preferred_element_type=jnp.float32)
    y3_ref[...] = y3

    @pl.when(i == 0)
    def _():
        st3_ref[...] = jnp.zeros_like(st3_ref)

    st3_ref[0:1, :] += jnp.sum(y3, axis=0, keepdims=True)
    st3_ref[1:2, :] += jnp.sum(y3 * y3, axis=0, keepdims=True)


def _run_d(y2, st2, gkp, bkp, w2):
    return pl.pallas_call(
        _d_body,
        grid=(_NBD,),
        in_specs=[
            pl.BlockSpec((_BD, _D2), lambda i: (i, 0)),
            pl.BlockSpec((8, 128), lambda i: (0, 0)),
            pl.BlockSpec((1, _D2), lambda i: (0, 0)),
            pl.BlockSpec((1, _D2), lambda i: (0, 0)),
            pl.BlockSpec((_D2, _CIN), lambda i: (0, 0)),
        ],
        out_specs=[
            pl.BlockSpec((_BD, _CIN), lambda i: (i, 0)),
            pl.BlockSpec((8, 128), lambda i: (0, 0)),
        ],
        out_shape=[
            jax.ShapeDtypeStruct((_N, _CIN), jnp.float32),
            jax.ShapeDtypeStruct((8, 128), jnp.float32),
        ],
    )(y2, st2, gkp, bkp, w2)


def _e_body(y3_ref, st3_ref, g2_ref, b2_ref, x_ref, out_ref):
    mean = st3_ref[0:1, :] * (1.0 / _N)
    ex2 = st3_ref[1:2, :] * (1.0 / _N)
    var = ex2 - mean * mean
    a = lax.rsqrt(var + _EPS) * g2_ref[...]
    c = b2_ref[...] - mean * a
    out_ref[...] = _leaky(y3_ref[...] * a + c) + x_ref[...]


def _run_e(y3, st3, g2v, b2v, x):
    return pl.pallas_call(
        _e_body,
        grid=(_NBD,),
        in_specs=[
            pl.BlockSpec((_BD, _CIN), lambda i: (i, 0)),
            pl.BlockSpec((8, 128), lambda i: (0, 0)),
            pl.BlockSpec((1, _CIN), lambda i: (0, 0)),
            pl.BlockSpec((1, _CIN), lambda i: (0, 0)),
            pl.BlockSpec((_BD, _CIN), lambda i: (i, 0)),
        ],
        out_specs=pl.BlockSpec((_BD, _CIN), lambda i: (i, 0)),
        out_shape=jax.ShapeDtypeStruct((_N, _CIN), jnp.float32),
    )(y3, st3, g2v, b2v, x)


_SEL_NP = np.zeros((768, 64), np.float32)
_T64_NP = np.zeros((3, 64), np.float32)
_TF_NP = np.zeros((_D2, 768), np.float32)
for _h in range(_H):
    for _c in range(4):
        _SEL_NP[48 * _h + _c, 4 * _h + _c] = 1.0
    for _c in range(3):
        _T64_NP[_c, 4 * _h + _c] = 1.0
    for _c in range(_D2):
        _TF_NP[_c, 48 * _h + 4 + _c] = 1.0


def kernel(x, pos, idx_neighboors, w_unary1, gamma1, beta1, kp_weight,
           K_points, gamma_kp, beta_kp, w_unary2, gamma2, beta2):
    idx_flat = idx_neighboors.reshape(-1).astype(jnp.int32)
    tab, st1 = _run_a(x, pos, w_unary1)
    g = _run_gather(tab, idx_flat)
    g2 = g.reshape(_N, _H * 48)
    eye16 = jnp.eye(_H, dtype=jnp.float32)
    blkK = jnp.zeros((4, _KP), jnp.float32).at[0:3, 0:_K].set(K_points.T)
    blk1 = jnp.zeros((4, _KP), jnp.float32).at[0:3, 0:_K].set(1.0)
    geo = jnp.concatenate([
        jnp.kron(eye16, -2.0 * blkK),
        jnp.kron(eye16, blk1),
        jnp.kron(eye16, -2.0 * blk1),
        jnp.tile(jnp.zeros((3, _KP), jnp.float32).at[:, 0:_K]
                 .set(2.0 * K_points.T), (1, _H)),
        jnp.tile(jnp.zeros((3, _KP), jnp.float32).at[:, 0:_K].set(1.0),
                 (1, _H)),
    ], axis=0)
    kn16 = (jnp.zeros((_KP,), jnp.float32)
            .at[0:_K].set(jnp.sum(K_points * K_points, axis=1))
            .at[_K].set(1e12))
    kn = jnp.tile(kn16, _H)[None]
    kpf = jnp.concatenate(
        [kp_weight.reshape(_K * _D2, _D2),
         jnp.zeros((_D2, _D2), jnp.float32)], axis=0)
    y2, st2 = _run_c(pos, g2, st1, gamma1[None], beta1[None],
                     jnp.asarray(_SEL_NP), jnp.asarray(_T64_NP),
                     jnp.asarray(_TF_NP), geo, kn, kpf)
    y3, st3 = _run_d(y2, st2, gamma_kp[None], beta_kp[None], w_unary2)
    return _run_e(y3, st3, gamma2[None], beta2[None], x)


# speedup vs baseline: 1.5230x; 1.0048x over previous
import functools

import jax
import jax.numpy as jnp
import numpy as np
from jax import lax
from jax.experimental import pallas as pl
from jax.experimental.pallas import tpu as pltpu
from jax.experimental.pallas import tpu_sc as plsc

_N = 50000
_H = 16
_K = 15
_D2 = 32
_CIN = 128
_PI = 0.04
_EPS = 1e-5

_BA = 400
_NBA = _N // _BA
_TROWS = (_NBA + 1) * _BA

_BC = 400
_NBC = _N // _BC
_KP = 16

_BD = 1000
_NBD = _N // _BD

_ROWS = _N * _H
_NW = 32
_PERW = _ROWS // _NW
_CHUNK = 1000
_NIT = _PERW // _CHUNK


def _leaky(v):
    return jnp.where(v >= 0, v, 0.2 * v)


def _a_body(x_ref, pos_ref, w1_ref, tab_ref, st_ref):
    i = pl.program_id(0)
    y1 = jnp.dot(x_ref[...], w1_ref[...], preferred_element_type=jnp.float32)
    real = i < _NBA
    row = jnp.concatenate(
        [pos_ref[...], jnp.zeros((_BA, 1), jnp.float32), y1,
         jnp.zeros((_BA, 12), jnp.float32)], axis=1)
    shadow = jnp.concatenate(
        [jnp.full((_BA, 3), 1e6, jnp.float32),
         jnp.zeros((_BA, 45), jnp.float32)], axis=1)
    tab_ref[...] = jnp.where(real, row, shadow)

    @pl.when(i == 0)
    def _():
        st_ref[...] = jnp.zeros_like(st_ref)

    @pl.when(real)
    def _():
        st_ref[0:1, 0:_D2] += jnp.sum(y1, axis=0, keepdims=True)
        st_ref[1:2, 0:_D2] += jnp.sum(y1 * y1, axis=0, keepdims=True)


def _run_a(x, pos, w1):
    return pl.pallas_call(
        _a_body,
        grid=(_NBA + 1,),
        in_specs=[
            pl.BlockSpec((_BA, _CIN), lambda i: (jnp.minimum(i, _NBA - 1), 0)),
            pl.BlockSpec((_BA, 3), lambda i: (jnp.minimum(i, _NBA - 1), 0)),
            pl.BlockSpec((_CIN, _D2), lambda i: (0, 0)),
        ],
        out_specs=[
            pl.BlockSpec((_BA, 48), lambda i: (i, 0)),
            pl.BlockSpec((8, 128), lambda i: (0, 0)),
        ],
        out_shape=[
            jax.ShapeDtypeStruct((_TROWS, 48), jnp.float32),
            jax.ShapeDtypeStruct((8, 128), jnp.float32),
        ],
    )(x, pos, w1)


def _gather_body(tab_hbm, idx_hbm, out_hbm, idx_v0, idx_v1, rows_v0,
                 rows_v1, sem0, sem1):
    wid = lax.axis_index("s") * 2 + lax.axis_index("c")
    base = wid * _PERW

    pltpu.sync_copy(idx_hbm.at[pl.ds(base, _CHUNK)], idx_v0)
    pltpu.async_copy(tab_hbm.at[idx_v0], rows_v0, sem0)

    def body(t, carry):
        off0 = base + (2 * t) * _CHUNK
        off1 = base + (2 * t + 1) * _CHUNK
        offn = base + (2 * t + 2) * _CHUNK
        pltpu.sync_copy(idx_hbm.at[pl.ds(off1, _CHUNK)], idx_v1)
        pltpu.async_copy(tab_hbm.at[idx_v1], rows_v1, sem1)
        pltpu.make_async_copy(tab_hbm.at[idx_v0], rows_v0, sem0).wait()
        pltpu.sync_copy(rows_v0, out_hbm.at[pl.ds(off0, _CHUNK)])
        pltpu.sync_copy(idx_hbm.at[pl.ds(offn, _CHUNK)], idx_v0)
        pltpu.async_copy(tab_hbm.at[idx_v0], rows_v0, sem0)
        pltpu.make_async_copy(tab_hbm.at[idx_v1], rows_v1, sem1).wait()
        pltpu.sync_copy(rows_v1, out_hbm.at[pl.ds(off1, _CHUNK)])
        return carry

    lax.fori_loop(0, _NIT // 2, body, 0)
    pltpu.make_async_copy(tab_hbm.at[idx_v0], rows_v0, sem0).wait()
    pltpu.sync_copy(rows_v0, out_hbm.at[pl.ds(base + (_NIT - 1) * _CHUNK,
                                              _CHUNK)])


def _run_gather(tab, idx_flat):
    mesh = plsc.VectorSubcoreMesh(core_axis_name="c", subcore_axis_name="s")
    f = functools.partial(
        pl.kernel,
        mesh=mesh,
        compiler_params=pltpu.CompilerParams(use_tc_tiling_on_sc=False),
        out_type=jax.ShapeDtypeStruct((_ROWS, 48), jnp.float32),
        scratch_types=[
            pltpu.VMEM((_CHUNK,), jnp.int32),
            pltpu.VMEM((_CHUNK,), jnp.int32),
            pltpu.VMEM((_CHUNK, 48), jnp.float32),
            pltpu.VMEM((_CHUNK, 48), jnp.float32),
            pltpu.SemaphoreType.DMA,
            pltpu.SemaphoreType.DMA,
        ],
    )(_gather_body)
    return f(tab, idx_flat)


def _c_body(pos_ref, g_ref, st1_ref, g1_ref, b1_ref, sel_ref, t64_ref,
            tf_ref, geo_ref, kn_ref, kpf_ref, y2_ref, st2_ref, wf_ref,
            wf2_ref):
    i = pl.program_id(0)
    mean = st1_ref[0:1, 0:_D2] * (1.0 / _N)
    ex2 = st1_ref[1:2, 0:_D2] * (1.0 / _N)
    var = ex2 - mean * mean
    a = lax.rsqrt(var + _EPS) * g1_ref[...]
    c = b1_ref[...] - mean * a
    qpos = pos_ref[...]
    g2v = g_ref[...]
    ktT = geo_ref[0:3, 0:_KP] * (-0.5)
    kn16 = kn_ref[0:1, 0:_KP]
    ws = []
    for h in range(_H):
        d = g_ref[:, 48 * h:48 * h + 3] - qpos
        sqn = jnp.sum(d * d, axis=1, keepdims=True)
        dt = (d[:, 0:1] * ktT[0:1] + d[:, 1:2] * ktT[1:2]
              + d[:, 2:3] * ktT[2:3])
        sqd_h = jnp.maximum(sqn - 2.0 * dt + kn16, 0.0)
        ws.append(jnp.maximum(1.0 - jnp.sqrt(sqd_h) * (1.0 / _PI), 0.0))
    w = jnp.concatenate(ws, axis=1)
    a768 = jnp.dot(a, tf_ref[...], preferred_element_type=jnp.float32)
    c768 = jnp.dot(c, tf_ref[...], preferred_element_type=jnp.float32)
    f_all = _leaky(g2v * a768 + c768)

    def _prod(h):
        f_h = f_all[:, 48 * h + 4:48 * h + 36]
        wexp = jnp.concatenate(
            [jnp.broadcast_to(w[:, _KP * h + k:_KP * h + k + 1], (_BC, _D2))
             for k in range(_KP)], axis=1)
        tile = jnp.concatenate([f_h] * _KP, axis=1)
        return wexp * tile

    wf_ref[...] = (_prod(0) + _prod(1)) + (_prod(2) + _prod(3))
    wf2_ref[...] = (_prod(4) + _prod(5)) + (_prod(6) + _prod(7))
    wf_ref[...] += (_prod(8) + _prod(9)) + (_prod(10) + _prod(11))
    wf2_ref[...] += (_prod(12) + _prod(13)) + (_prod(14) + _prod(15))
    y2 = jnp.dot(wf_ref[...] + wf2_ref[...], kpf_ref[...],
                 preferred_element_type=jnp.float32)
    y2_ref[...] = y2

    @pl.when(i == 0)
    def _():
        st2_ref[...] = jnp.zeros_like(st2_ref)

    st2_ref[0:1, 0:_D2] += jnp.sum(y2, axis=0, keepdims=True)
    st2_ref[1:2, 0:_D2] += jnp.sum(y2 * y2, axis=0, keepdims=True)


def _run_c(pos, g2, st1, g1, b1, sel, t64, tf, geo, kn, kpf):
    return pl.pallas_call(
        _c_body,
        grid=(_NBC,),
        in_specs=[
            pl.BlockSpec((_BC, 3), lambda i: (i, 0)),
            pl.BlockSpec((_BC, _H * 48), lambda i: (i, 0)),
            pl.BlockSpec((8, 128), lambda i: (0, 0)),
            pl.BlockSpec((1, _D2), lambda i: (0, 0)),
            pl.BlockSpec((1, _D2), lambda i: (0, 0)),
            pl.BlockSpec((768, 64), lambda i: (0, 0)),
            pl.BlockSpec((3, 64), lambda i: (0, 0)),
            pl.BlockSpec((_D2, 768), lambda i: (0, 0)),
            pl.BlockSpec((198, _KP * _H), lambda i: (0, 0)),
            pl.BlockSpec((1, _KP * _H), lambda i: (0, 0)),
            pl.BlockSpec((_KP * _D2, _D2), lambda i: (0, 0)),
        ],
        out_specs=[
            pl.BlockSpec((_BC, _D2), lambda i: (i, 0)),
            pl.BlockSpec((8, 128), lambda i: (0, 0)),
        ],
        out_shape=[
            jax.ShapeDtypeStruct((_N, _D2), jnp.float32),
            jax.ShapeDtypeStruct((8, 128), jnp.float32),
        ],
        scratch_shapes=[pltpu.VMEM((_BC, _KP * _D2), jnp.float32),
                        pltpu.VMEM((_BC, _KP * _D2), jnp.float32)],
    )(pos, g2, st1, g1, b1, sel, t64, tf, geo, kn, kpf)


def _d_body(y2_ref, st2_ref, gkp_ref, bkp_ref, w2_ref, y3_ref, st3_ref):
    i = pl.program_id(0)
    mean = st2_ref[0:1, 0:_D2] * (1.0 / _N)
    ex2 = st2_ref[1:2, 0:_D2] * (1.0 / _N)
    var = ex2 - mean * mean
    a = lax.rsqrt(var + _EPS) * gkp_ref[...]
    c = bkp_ref[...] - mean * a
    h2 = _leaky(y2_ref[...] * a + c)
    y3 = jnp.dot(h2, w2_ref[...], preferred_element_type=jnp.float32)
    y3_ref[...] = y3

    @pl.when(i == 0)
    def _():
        st3_ref[...] = jnp.zeros_like(st3_ref)

    st3_ref[0:1, :] += jnp.sum(y3, axis=0, keepdims=True)
    st3_ref[1:2, :] += jnp.sum(y3 * y3, axis=0, keepdims=True)


def _run_d(y2, st2, gkp, bkp, w2):
    return pl.pallas_call(
        _d_body,
        grid=(_NBD,),
        in_specs=[
            pl.BlockSpec((_BD, _D2), lambda i: (i, 0)),
            pl.BlockSpec((8, 128), lambda i: (0, 0)),
            pl.BlockSpec((1, _D2), lambda i: (0, 0)),
            pl.BlockSpec((1, _D2), lambda i: (0, 0)),
            pl.BlockSpec((_D2, _CIN), lambda i: (0, 0)),
        ],
        out_specs=[
            pl.BlockSpec((_BD, _CIN), lambda i: (i, 0)),
            pl.BlockSpec((8, 128), lambda i: (0, 0)),
        ],
        out_shape=[
            jax.ShapeDtypeStruct((_N, _CIN), jnp.float32),
            jax.ShapeDtypeStruct((8, 128), jnp.float32),
        ],
    )(y2, st2, gkp, bkp, w2)


def _e_body(y3_ref, st3_ref, g2_ref, b2_ref, x_ref, out_ref):
    mean = st3_ref[0:1, :] * (1.0 / _N)
    ex2 = st3_ref[1:2, :] * (1.0 / _N)
    var = ex2 - mean * mean
    a = lax.rsqrt(var + _EPS) * g2_ref[...]
    c = b2_ref[...] - mean * a
    out_ref[...] = _leaky(y3_ref[...] * a + c) + x_ref[...]


def _run_e(y3, st3, g2v, b2v, x):
    return pl.pallas_call(
        _e_body,
        grid=(_NBD,),
        in_specs=[
            pl.BlockSpec((_BD, _CIN), lambda i: (i, 0)),
            pl.BlockSpec((8, 128), lambda i: (0, 0)),
            pl.BlockSpec((1, _CIN), lambda i: (0, 0)),
            pl.BlockSpec((1, _CIN), lambda i: (0, 0)),
            pl.BlockSpec((_BD, _CIN), lambda i: (i, 0)),
        ],
        out_specs=pl.BlockSpec((_BD, _CIN), lambda i: (i, 0)),
        out_shape=jax.ShapeDtypeStruct((_N, _CIN), jnp.float32),
    )(y3, st3, g2v, b2v, x)


_SEL_NP = np.zeros((768, 64), np.float32)
_T64_NP = np.zeros((3, 64), np.float32)
_TF_NP = np.zeros((_D2, 768), np.float32)
for _h in range(_H):
    for _c in range(4):
        _SEL_NP[48 * _h + _c, 4 * _h + _c] = 1.0
    for _c in range(3):
        _T64_NP[_c, 4 * _h + _c] = 1.0
    for _c in range(_D2):
        _TF_NP[_c, 48 * _h + 4 + _c] = 1.0


def kernel(x, pos, idx_neighboors, w_unary1, gamma1, beta1, kp_weight,
           K_points, gamma_kp, beta_kp, w_unary2, gamma2, beta2):
    idx_flat = idx_neighboors.reshape(-1).astype(jnp.int32)
    tab, st1 = _run_a(x, pos, w_unary1)
    g = _run_gather(tab, idx_flat)
    g2 = g.reshape(_N, _H * 48)
    eye16 = jnp.eye(_H, dtype=jnp.float32)
    blkK = jnp.zeros((4, _KP), jnp.float32).at[0:3, 0:_K].set(K_points.T)
    blk1 = jnp.zeros((4, _KP), jnp.float32).at[0:3, 0:_K].set(1.0)
    geo = jnp.concatenate([
        jnp.kron(eye16, -2.0 * blkK),
        jnp.kron(eye16, blk1),
        jnp.kron(eye16, -2.0 * blk1),
        jnp.tile(jnp.zeros((3, _KP), jnp.float32).at[:, 0:_K]
                 .set(2.0 * K_points.T), (1, _H)),
        jnp.tile(jnp.zeros((3, _KP), jnp.float32).at[:, 0:_K].set(1.0),
                 (1, _H)),
    ], axis=0)
    kn16 = (jnp.zeros((_KP,), jnp.float32)
            .at[0:_K].set(jnp.sum(K_points * K_points, axis=1))
            .at[_K].set(1e12))
    kn = jnp.tile(kn16, _H)[None]
    kpf = jnp.concatenate(
        [kp_weight.reshape(_K * _D2, _D2),
         jnp.zeros((_D2, _D2), jnp.float32)], axis=0)
    y2, st2 = _run_c(pos, g2, st1, gamma1[None], beta1[None],
                     jnp.asarray(_SEL_NP), jnp.asarray(_T64_NP),
                     jnp.asarray(_TF_NP), geo, kn, kpf)
    y3, st3 = _run_d(y2, st2, gamma_kp[None], beta_kp[None], w_unary2)
    return _run_e(y3, st3, gamma2[None], beta2[None], x)
